# Initial kernel scaffold; baseline (speedup 1.0000x reference)
import functools

import jax
import jax.numpy as jnp
from jax import lax
from jax.experimental import pallas as pl
from jax.experimental.pallas import tpu as pltpu
from jax.experimental.pallas import tpu_sc as plsc

_ROWS = 4096
_COLS = 50
_EMBED = 64
_B = _ROWS * _COLS
_NW = 32
_BPW = _B // _NW
_CHUNK = 800
_NCHUNK = _BPW // _CHUNK

_mesh = plsc.VectorSubcoreMesh(core_axis_name="c", subcore_axis_name="s")


@functools.partial(
    pl.kernel,
    mesh=_mesh,
    out_type=jax.ShapeDtypeStruct((_B, _EMBED), jnp.float32),
    scratch_types=[
        pltpu.VMEM((_CHUNK,), jnp.int32),
        pltpu.VMEM((_CHUNK, _EMBED), jnp.float32),
        pltpu.SemaphoreType.DMA,
    ],
)
def _sc_gather(x_hbm, enc_hbm, out_hbm, idx_v, rows_v, sem):
    wid = lax.axis_index("s") * 2 + lax.axis_index("c")
    base = wid * _BPW

    def body(i, carry):
        off = base + i * _CHUNK
        pltpu.sync_copy(x_hbm.at[pl.ds(off, _CHUNK)], idx_v)
        pltpu.async_copy(enc_hbm.at[idx_v], rows_v, sem).wait()
        pltpu.sync_copy(rows_v, out_hbm.at[pl.ds(off, _CHUNK)])
        return carry

    lax.fori_loop(0, _NCHUNK, body, 0)


def kernel(x, encoding):
    out = _sc_gather(x.reshape(_B), encoding)
    return out.reshape(_ROWS, _COLS, _EMBED)


# speedup vs baseline: 4.8895x; 4.8895x over previous
import functools

import jax
import jax.numpy as jnp
from jax import lax
from jax.experimental import pallas as pl
from jax.experimental.pallas import tpu as pltpu
from jax.experimental.pallas import tpu_sc as plsc

_ROWS = 4096
_COLS = 50
_EMBED = 64
_B = _ROWS * _COLS
_NW = 32
_BPW = _B // _NW
_CHUNK = 800
_NCHUNK = _BPW // _CHUNK

_mesh = plsc.VectorSubcoreMesh(core_axis_name="c", subcore_axis_name="s")


@functools.partial(
    pl.kernel,
    mesh=_mesh,
    compiler_params=pltpu.CompilerParams(use_tc_tiling_on_sc=False),
    out_type=jax.ShapeDtypeStruct((_B, _EMBED), jnp.float32),
    scratch_types=[
        pltpu.VMEM((_CHUNK,), jnp.int32),
        pltpu.VMEM((_CHUNK, _EMBED), jnp.float32),
        pltpu.SemaphoreType.DMA,
    ],
)
def _sc_gather(x_hbm, enc_hbm, out_hbm, idx_v, rows_v, sem):
    wid = lax.axis_index("s") * 2 + lax.axis_index("c")
    base = wid * _BPW

    def body(i, carry):
        off = base + i * _CHUNK
        pltpu.sync_copy(x_hbm.at[pl.ds(off, _CHUNK)], idx_v)
        pltpu.async_copy(enc_hbm.at[idx_v], rows_v, sem).wait()
        pltpu.sync_copy(rows_v, out_hbm.at[pl.ds(off, _CHUNK)])
        return carry

    lax.fori_loop(0, _NCHUNK, body, 0)


def kernel(x, encoding):
    out = _sc_gather(x.reshape(_B), encoding)
    return out.reshape(_ROWS, _COLS, _EMBED)
